# baseline (speedup 1.0000x reference)
import functools

import jax
import jax.numpy as jnp
from jax import lax
from jax.experimental import pallas as pl
from jax.experimental.pallas import tpu as pltpu
from jax.experimental.pallas import tpu_sc as plsc

N = 10000
E = 320000
D = 128
G = 64

NC = 2
NS = 16
NW = NC * NS
CH = 100
C = 100
SUP = 5
SCH = CH // SUP
NPAD = 10240
ZR = 80
RPT = NPAD // NS

BM = 1000
NB = N // BM


NZ = RPT // ZR


def _sc_agg_body(src_hbm, dst_hbm, h_hbm, z_hbm, out_hbm,
                 agg_sh, idx_s0, idx_d0, idx_s1, idx_d1,
                 rows0, rows1,
                 sem0, sem1, semI0, semI1, semR0, semR1, semW0, semW1,
                 semS0, semS1):
    c = lax.axis_index("c")
    s = lax.axis_index("s")
    wid = c * NS + s
    zb0 = rows0.at[pl.ds(0, ZR)]
    zb1 = rows1.at[pl.ds(0, ZR)]

    pltpu.async_copy(z_hbm, zb0, semR0)
    pltpu.async_copy(src_hbm.at[wid, 0], idx_s0, semI0)
    pltpu.async_copy(dst_hbm.at[wid, 0], idx_d0, semI0)
    pltpu.make_async_copy(z_hbm, zb0, semR0).wait()
    for k in range(NZ):
        pltpu.async_copy(zb0, agg_sh.at[pl.ds((s * NZ + k) * ZR, ZR)], semW0)
    pltpu.make_async_copy(src_hbm.at[wid, 0], idx_s0, semI0).wait()
    pltpu.make_async_copy(dst_hbm.at[wid, 0], idx_d0, semI0).wait()
    for k in range(NZ):
        pltpu.make_async_copy(zb0, agg_sh.at[pl.ds(0, ZR)], semW0).wait()
    pltpu.async_copy(h_hbm.at[idx_s0.at[0]], rows0, sem0)
    pltpu.async_copy(h_hbm.at[idx_s0.at[1]], rows1, sem1)
    plsc.subcore_barrier()

    idx_pairs = [(idx_s0, idx_d0, semI0), (idx_s1, idx_d1, semI1)]
    for t in range(SUP):
        cs, cd, csem = idx_pairs[t % 2]
        if t + 1 < SUP:
            ns, nd, nsem = idx_pairs[(t + 1) % 2]
            pltpu.async_copy(src_hbm.at[wid, t + 1], ns, nsem)
            pltpu.async_copy(dst_hbm.at[wid, t + 1], nd, nsem)

        def pair(g, carry, cs=cs, cd=cd):
            j = 2 * g
            pltpu.make_async_copy(h_hbm.at[cs.at[j]], rows0, sem0).wait()
            pltpu.async_copy(rows0, agg_sh.at[cd.at[j]], semS0, add=True)
            pltpu.make_async_copy(h_hbm.at[cs.at[j + 1]], rows1, sem1).wait()
            pltpu.async_copy(rows1, agg_sh.at[cd.at[j + 1]], semS1, add=True)
            pltpu.make_async_copy(rows0, agg_sh.at[cd.at[j]], semS0).wait()
            pltpu.async_copy(h_hbm.at[cs.at[j + 2]], rows0, sem0)
            pltpu.make_async_copy(rows1, agg_sh.at[cd.at[j + 1]],
                                  semS1).wait()
            pltpu.async_copy(h_hbm.at[cs.at[j + 3]], rows1, sem1)
            return carry

        lax.fori_loop(0, SCH // 2 - 1, pair, 0)
        pltpu.make_async_copy(h_hbm.at[cs.at[SCH - 2]], rows0, sem0).wait()
        pltpu.async_copy(rows0, agg_sh.at[cd.at[SCH - 2]], semS0, add=True)
        pltpu.make_async_copy(h_hbm.at[cs.at[SCH - 1]], rows1, sem1).wait()
        pltpu.async_copy(rows1, agg_sh.at[cd.at[SCH - 1]], semS1, add=True)
        pltpu.make_async_copy(rows0, agg_sh.at[cd.at[SCH - 2]], semS0).wait()
        if t + 1 < SUP:
            pltpu.make_async_copy(src_hbm.at[wid, t + 1], ns, nsem).wait()
            pltpu.make_async_copy(dst_hbm.at[wid, t + 1], nd, nsem).wait()
            pltpu.async_copy(h_hbm.at[ns.at[0]], rows0, sem0)
        pltpu.make_async_copy(rows1, agg_sh.at[cd.at[SCH - 1]], semS1).wait()
        if t + 1 < SUP:
            pltpu.async_copy(h_hbm.at[ns.at[1]], rows1, sem1)
    plsc.subcore_barrier()

    bufs = (zb0, zb1)
    rsems = (semR0, semR1)
    wsems = (semW0, semW1)
    pltpu.async_copy(agg_sh.at[pl.ds(s * NZ * ZR, ZR)], zb0, semR0)
    for k in range(NZ):
        kb = k % 2
        cur = bufs[kb]
        r0 = (s * NZ + k) * ZR
        pltpu.make_async_copy(agg_sh.at[pl.ds(r0, ZR)], cur, rsems[kb]).wait()
        if k >= 1:
            pltpu.make_async_copy(bufs[1 - kb], out_hbm.at[pl.ds(0, ZR)],
                                  wsems[1 - kb]).wait()
        if k + 1 < NZ:
            r1 = (s * NZ + k + 1) * ZR
            pltpu.async_copy(agg_sh.at[pl.ds(r1, ZR)], bufs[1 - kb],
                             rsems[1 - kb])
        pltpu.async_copy(cur, out_hbm.at[pl.ds(c * NPAD + r0, ZR)], wsems[kb])
    pltpu.make_async_copy(bufs[(NZ - 1) % 2], out_hbm.at[pl.ds(0, ZR)],
                          wsems[(NZ - 1) % 2]).wait()


_sc_agg = functools.partial(
    pl.kernel,
    mesh=plsc.VectorSubcoreMesh(core_axis_name="c", subcore_axis_name="s"),
    out_type=jax.ShapeDtypeStruct((2 * NPAD, D), jnp.float32),
    scratch_types=[
        pltpu.VMEM_SHARED((NPAD, D), jnp.float32),
        pltpu.VMEM((SCH, C), jnp.int32),
        pltpu.VMEM((SCH, C), jnp.int32),
        pltpu.VMEM((SCH, C), jnp.int32),
        pltpu.VMEM((SCH, C), jnp.int32),
        pltpu.VMEM((C, D), jnp.float32),
        pltpu.VMEM((C, D), jnp.float32),
        pltpu.SemaphoreType.DMA,
        pltpu.SemaphoreType.DMA,
        pltpu.SemaphoreType.DMA,
        pltpu.SemaphoreType.DMA,
        pltpu.SemaphoreType.DMA,
        pltpu.SemaphoreType.DMA,
        pltpu.SemaphoreType.DMA,
        pltpu.SemaphoreType.DMA,
        pltpu.SemaphoreType.DMA,
        pltpu.SemaphoreType.DMA,
    ],
)(_sc_agg_body)


def _phase0(h_ref, p_ref, w_ref, b_ref, eps_ref, y_s, st_s, i):
    z = h_ref[...] * (1.0 + eps_ref[0, 0]) + p_ref[0] + p_ref[1]
    y = jnp.dot(z, w_ref[...], preferred_element_type=jnp.float32) + b_ref[...]
    y_s[pl.ds(i * BM, BM), :] = y
    s1 = jnp.sum(y, axis=0, keepdims=True)
    s2 = jnp.sum(y * y, axis=0, keepdims=True)
    upd = jnp.concatenate([s1, s2, jnp.zeros((6, D), jnp.float32)], axis=0)

    @pl.when(i == 0)
    def _():
        st_s[...] = upd

    @pl.when(i > 0)
    def _():
        st_s[...] = st_s[...] + upd


def _bn_relu_from_scratch(y_s, st_s, g_ref, be_ref, j):
    y = y_s[pl.ds(j * BM, BM), :]
    mean = st_s[0:1, :] * (1.0 / N)
    ex2 = st_s[1:2, :] * (1.0 / N)
    var = ex2 - mean * mean
    inv = lax.rsqrt(var + 1e-5) * g_ref[...]
    return jnp.maximum((y - mean) * inv + be_ref[...], 0.0)


def _layer_body(h_ref, p_ref, w_ref, b_ref, eps_ref, g_ref, be_ref,
                o_ref, y_s, st_s):
    i = pl.program_id(0)

    @pl.when(i < NB)
    def _():
        _phase0(h_ref, p_ref, w_ref, b_ref, eps_ref, y_s, st_s, i)

    @pl.when(i >= NB)
    def _():
        o_ref[...] = _bn_relu_from_scratch(y_s, st_s, g_ref, be_ref, i - NB)


_COMMON_SPECS = [
    pl.BlockSpec((BM, D), lambda i: (jnp.minimum(i, NB - 1), 0)),
    pl.BlockSpec((2, BM, D), lambda i: (0, jnp.minimum(i, NB - 1), 0)),
    pl.BlockSpec((D, D), lambda i: (0, 0)),
    pl.BlockSpec((1, D), lambda i: (0, 0)),
    pl.BlockSpec(memory_space=pltpu.SMEM),
    pl.BlockSpec((1, D), lambda i: (0, 0)),
    pl.BlockSpec((1, D), lambda i: (0, 0)),
]


def _layer(h, parts, w, b2, eps2, g2, be2):
    return pl.pallas_call(
        _layer_body,
        grid=(2 * NB,),
        in_specs=_COMMON_SPECS,
        out_specs=pl.BlockSpec((BM, D), lambda i: (i % NB, 0)),
        out_shape=jax.ShapeDtypeStruct((N, D), jnp.float32),
        scratch_shapes=[
            pltpu.VMEM((N, D), jnp.float32),
            pltpu.VMEM((8, D), jnp.float32),
        ],
    )(h, parts, w, b2, eps2, g2, be2)


def _layer_pool_body(h_ref, p_ref, w_ref, b_ref, eps_ref, g_ref, be_ref,
                     bt_ref, o_ref, y_s, st_s, sums, cnts):
    i = pl.program_id(0)

    @pl.when(i < NB)
    def _():
        _phase0(h_ref, p_ref, w_ref, b_ref, eps_ref, y_s, st_s, i)

    @pl.when(i >= NB)
    def _():
        h = _bn_relu_from_scratch(y_s, st_s, g_ref, be_ref, i - NB)
        bvec = bt_ref[0, 0, :]
        oh = (bvec[:, None] == lax.broadcasted_iota(jnp.int32, (BM, G), 1))
        oh = oh.astype(jnp.float32)
        psum = lax.dot_general(oh, h, (((0,), (0,)), ((), ())),
                               preferred_element_type=jnp.float32)
        pcnt = jnp.broadcast_to(jnp.sum(oh, axis=0)[:, None], (G, D))

        @pl.when(i == NB)
        def _():
            sums[...] = psum
            cnts[...] = pcnt

        @pl.when(i > NB)
        def _():
            sums[...] = sums[...] + psum
            cnts[...] = cnts[...] + pcnt

        @pl.when(i == 2 * NB - 1)
        def _():
            o_ref[...] = sums[...] / jnp.maximum(cnts[...], 1.0)


def _layer_pool(h, parts, w, b2, eps2, g2, be2, batch_r):
    return pl.pallas_call(
        _layer_pool_body,
        grid=(2 * NB,),
        in_specs=_COMMON_SPECS + [
            pl.BlockSpec((1, 1, BM), lambda i: (jnp.maximum(i - NB, 0), 0, 0)),
        ],
        out_specs=pl.BlockSpec((G, D), lambda i: (0, 0)),
        out_shape=jax.ShapeDtypeStruct((G, D), jnp.float32),
        scratch_shapes=[
            pltpu.VMEM((N, D), jnp.float32),
            pltpu.VMEM((8, D), jnp.float32),
            pltpu.VMEM((G, D), jnp.float32),
            pltpu.VMEM((G, D), jnp.float32),
        ],
    )(h, parts, w, b2, eps2, g2, be2, batch_r)


def kernel(edge_index, x, batch, W0, b0, eps0, g0, be0,
           W1, b1, eps1, g1, be1, W2, b2, eps2, g2, be2):
    src = edge_index[0].reshape(NW, SUP, SCH, C)
    dst = edge_index[1].reshape(NW, SUP, SCH, C)
    zeros_rows = jnp.zeros((ZR, D), jnp.float32)
    batch_r = batch.reshape(NB, 1, BM)

    params = [(W0, b0, eps0, g0, be0),
              (W1, b1, eps1, g1, be1),
              (W2, b2, eps2, g2, be2)]
    h = x
    for li, (W, b, eps, g, be) in enumerate(params):
        parts = _sc_agg(src, dst, h, zeros_rows).reshape(2, NPAD, D)
        args = (h, parts, W, b.reshape(1, D), eps.reshape(1, 1),
                g.reshape(1, D), be.reshape(1, D))
        if li < 2:
            h = _layer(*args)
        else:
            out = _layer_pool(*args, batch_r)
    return out


# speedup vs baseline: 1.2739x; 1.2739x over previous
import functools

import jax
import jax.numpy as jnp
from jax import lax
from jax.experimental import pallas as pl
from jax.experimental.pallas import tpu as pltpu
from jax.experimental.pallas import tpu_sc as plsc

N = 10000
E = 320000
D = 128
G = 64

NC = 2
NS = 16
NW = NC * NS
CH = 80
C = 125
SUP = 4
SCH = CH // SUP
NPAD = 10240
ZR = 80
RPT = NPAD // NS

BM = 1000
NB = N // BM


NZ = RPT // ZR


def _sc_agg_body(src_hbm, dst_hbm, h_hbm, z_hbm, out_hbm,
                 agg_sh, idx_s0, idx_d0, idx_s1, idx_d1,
                 rows0, rows1,
                 sem0, sem1, semI0, semI1, semR0, semR1, semW0, semW1):
    c = lax.axis_index("c")
    s = lax.axis_index("s")
    wid = c * NS + s
    zb0 = rows0.at[pl.ds(0, ZR)]
    zb1 = rows1.at[pl.ds(0, ZR)]

    pltpu.async_copy(z_hbm, zb0, semR0)
    pltpu.async_copy(src_hbm.at[wid, 0], idx_s0, semI0)
    pltpu.async_copy(dst_hbm.at[wid, 0], idx_d0, semI0)
    pltpu.make_async_copy(z_hbm, zb0, semR0).wait()
    for k in range(NZ):
        pltpu.async_copy(zb0, agg_sh.at[pl.ds((s * NZ + k) * ZR, ZR)], semW0)
    pltpu.make_async_copy(src_hbm.at[wid, 0], idx_s0, semI0).wait()
    pltpu.make_async_copy(dst_hbm.at[wid, 0], idx_d0, semI0).wait()
    for k in range(NZ):
        pltpu.make_async_copy(zb0, agg_sh.at[pl.ds(0, ZR)], semW0).wait()
    pltpu.async_copy(h_hbm.at[idx_s0.at[0]], rows0, sem0)
    plsc.subcore_barrier()

    idx_pairs = [(idx_s0, idx_d0, semI0), (idx_s1, idx_d1, semI1)]
    for t in range(SUP):
        cs, cd, csem = idx_pairs[t % 2]
        if t > 0:
            pltpu.make_async_copy(src_hbm.at[wid, t], cs, csem).wait()
            pltpu.make_async_copy(dst_hbm.at[wid, t], cd, csem).wait()
        if t + 1 < SUP:
            ns, nd, nsem = idx_pairs[(t + 1) % 2]
            pltpu.async_copy(src_hbm.at[wid, t + 1], ns, nsem)
            pltpu.async_copy(dst_hbm.at[wid, t + 1], nd, nsem)

        def pair(g, carry, cs=cs, cd=cd):
            j = 2 * g
            pltpu.async_copy(h_hbm.at[cs.at[j + 1]], rows1, sem1)
            pltpu.make_async_copy(h_hbm.at[cs.at[j]], rows0, sem0).wait()
            pltpu.sync_copy(rows0, agg_sh.at[cd.at[j]], add=True)
            pltpu.async_copy(h_hbm.at[cs.at[j + 2]], rows0, sem0)
            pltpu.make_async_copy(h_hbm.at[cs.at[j + 1]], rows1, sem1).wait()
            pltpu.sync_copy(rows1, agg_sh.at[cd.at[j + 1]], add=True)
            return carry

        if t > 0:
            pltpu.async_copy(h_hbm.at[cs.at[0]], rows0, sem0)
        lax.fori_loop(0, (SCH - 1) // 2, pair, 0)
        if SCH % 2 == 1:
            pltpu.make_async_copy(h_hbm.at[cs.at[SCH - 1]], rows0,
                                  sem0).wait()
            pltpu.sync_copy(rows0, agg_sh.at[cd.at[SCH - 1]], add=True)
        else:
            pltpu.async_copy(h_hbm.at[cs.at[SCH - 1]], rows1, sem1)
            pltpu.make_async_copy(h_hbm.at[cs.at[SCH - 2]], rows0,
                                  sem0).wait()
            pltpu.sync_copy(rows0, agg_sh.at[cd.at[SCH - 2]], add=True)
            pltpu.make_async_copy(h_hbm.at[cs.at[SCH - 1]], rows1,
                                  sem1).wait()
            pltpu.sync_copy(rows1, agg_sh.at[cd.at[SCH - 1]], add=True)
    plsc.subcore_barrier()

    bufs = (zb0, zb1)
    rsems = (semR0, semR1)
    wsems = (semW0, semW1)
    pltpu.async_copy(agg_sh.at[pl.ds(s * NZ * ZR, ZR)], zb0, semR0)
    for k in range(NZ):
        kb = k % 2
        cur = bufs[kb]
        r0 = (s * NZ + k) * ZR
        pltpu.make_async_copy(agg_sh.at[pl.ds(r0, ZR)], cur, rsems[kb]).wait()
        if k >= 1:
            pltpu.make_async_copy(bufs[1 - kb], out_hbm.at[pl.ds(0, ZR)],
                                  wsems[1 - kb]).wait()
        if k + 1 < NZ:
            r1 = (s * NZ + k + 1) * ZR
            pltpu.async_copy(agg_sh.at[pl.ds(r1, ZR)], bufs[1 - kb],
                             rsems[1 - kb])
        pltpu.async_copy(cur, out_hbm.at[pl.ds(c * NPAD + r0, ZR)], wsems[kb])
    pltpu.make_async_copy(bufs[(NZ - 1) % 2], out_hbm.at[pl.ds(0, ZR)],
                          wsems[(NZ - 1) % 2]).wait()


_sc_agg = functools.partial(
    pl.kernel,
    mesh=plsc.VectorSubcoreMesh(core_axis_name="c", subcore_axis_name="s"),
    out_type=jax.ShapeDtypeStruct((2 * NPAD, D), jnp.float32),
    scratch_types=[
        pltpu.VMEM_SHARED((NPAD, D), jnp.float32),
        pltpu.VMEM((SCH, C), jnp.int32),
        pltpu.VMEM((SCH, C), jnp.int32),
        pltpu.VMEM((SCH, C), jnp.int32),
        pltpu.VMEM((SCH, C), jnp.int32),
        pltpu.VMEM((C, D), jnp.float32),
        pltpu.VMEM((C, D), jnp.float32),
        pltpu.SemaphoreType.DMA,
        pltpu.SemaphoreType.DMA,
        pltpu.SemaphoreType.DMA,
        pltpu.SemaphoreType.DMA,
        pltpu.SemaphoreType.DMA,
        pltpu.SemaphoreType.DMA,
        pltpu.SemaphoreType.DMA,
        pltpu.SemaphoreType.DMA,
    ],
)(_sc_agg_body)


def _phase0(h_ref, p_ref, w_ref, b_ref, eps_ref, y_s, st_s, i):
    z = h_ref[...] * (1.0 + eps_ref[0, 0]) + p_ref[0] + p_ref[1]
    y = jnp.dot(z, w_ref[...], preferred_element_type=jnp.float32) + b_ref[...]
    y_s[pl.ds(i * BM, BM), :] = y
    s1 = jnp.sum(y, axis=0, keepdims=True)
    s2 = jnp.sum(y * y, axis=0, keepdims=True)
    upd = jnp.concatenate([s1, s2, jnp.zeros((6, D), jnp.float32)], axis=0)

    @pl.when(i == 0)
    def _():
        st_s[...] = upd

    @pl.when(i > 0)
    def _():
        st_s[...] = st_s[...] + upd


def _bn_relu_from_scratch(y_s, st_s, g_ref, be_ref, j):
    y = y_s[pl.ds(j * BM, BM), :]
    mean = st_s[0:1, :] * (1.0 / N)
    ex2 = st_s[1:2, :] * (1.0 / N)
    var = ex2 - mean * mean
    inv = lax.rsqrt(var + 1e-5) * g_ref[...]
    return jnp.maximum((y - mean) * inv + be_ref[...], 0.0)


def _layer_body(h_ref, p_ref, w_ref, b_ref, eps_ref, g_ref, be_ref,
                o_ref, y_s, st_s):
    i = pl.program_id(0)

    @pl.when(i < NB)
    def _():
        _phase0(h_ref, p_ref, w_ref, b_ref, eps_ref, y_s, st_s, i)

    @pl.when(i >= NB)
    def _():
        o_ref[...] = _bn_relu_from_scratch(y_s, st_s, g_ref, be_ref, i - NB)


_COMMON_SPECS = [
    pl.BlockSpec((BM, D), lambda i: (jnp.minimum(i, NB - 1), 0)),
    pl.BlockSpec((2, BM, D), lambda i: (0, jnp.minimum(i, NB - 1), 0)),
    pl.BlockSpec((D, D), lambda i: (0, 0)),
    pl.BlockSpec((1, D), lambda i: (0, 0)),
    pl.BlockSpec(memory_space=pltpu.SMEM),
    pl.BlockSpec((1, D), lambda i: (0, 0)),
    pl.BlockSpec((1, D), lambda i: (0, 0)),
]


def _layer(h, parts, w, b2, eps2, g2, be2):
    return pl.pallas_call(
        _layer_body,
        grid=(2 * NB,),
        in_specs=_COMMON_SPECS,
        out_specs=pl.BlockSpec((BM, D), lambda i: (i % NB, 0)),
        out_shape=jax.ShapeDtypeStruct((N, D), jnp.float32),
        scratch_shapes=[
            pltpu.VMEM((N, D), jnp.float32),
            pltpu.VMEM((8, D), jnp.float32),
        ],
    )(h, parts, w, b2, eps2, g2, be2)


def _layer_pool_body(h_ref, p_ref, w_ref, b_ref, eps_ref, g_ref, be_ref,
                     bt_ref, o_ref, y_s, st_s, sums, cnts):
    i = pl.program_id(0)

    @pl.when(i < NB)
    def _():
        _phase0(h_ref, p_ref, w_ref, b_ref, eps_ref, y_s, st_s, i)

    @pl.when(i >= NB)
    def _():
        h = _bn_relu_from_scratch(y_s, st_s, g_ref, be_ref, i - NB)
        bvec = bt_ref[0, 0, :]
        oh = (bvec[:, None] == lax.broadcasted_iota(jnp.int32, (BM, G), 1))
        oh = oh.astype(jnp.float32)
        psum = lax.dot_general(oh, h, (((0,), (0,)), ((), ())),
                               preferred_element_type=jnp.float32)
        pcnt = jnp.broadcast_to(jnp.sum(oh, axis=0)[:, None], (G, D))

        @pl.when(i == NB)
        def _():
            sums[...] = psum
            cnts[...] = pcnt

        @pl.when(i > NB)
        def _():
            sums[...] = sums[...] + psum
            cnts[...] = cnts[...] + pcnt

        @pl.when(i == 2 * NB - 1)
        def _():
            o_ref[...] = sums[...] / jnp.maximum(cnts[...], 1.0)


def _layer_pool(h, parts, w, b2, eps2, g2, be2, batch_r):
    return pl.pallas_call(
        _layer_pool_body,
        grid=(2 * NB,),
        in_specs=_COMMON_SPECS + [
            pl.BlockSpec((1, 1, BM), lambda i: (jnp.maximum(i - NB, 0), 0, 0)),
        ],
        out_specs=pl.BlockSpec((G, D), lambda i: (0, 0)),
        out_shape=jax.ShapeDtypeStruct((G, D), jnp.float32),
        scratch_shapes=[
            pltpu.VMEM((N, D), jnp.float32),
            pltpu.VMEM((8, D), jnp.float32),
            pltpu.VMEM((G, D), jnp.float32),
            pltpu.VMEM((G, D), jnp.float32),
        ],
    )(h, parts, w, b2, eps2, g2, be2, batch_r)


def kernel(edge_index, x, batch, W0, b0, eps0, g0, be0,
           W1, b1, eps1, g1, be1, W2, b2, eps2, g2, be2):
    src = edge_index[0].reshape(NW, SUP, SCH, C)
    dst = edge_index[1].reshape(NW, SUP, SCH, C)
    zeros_rows = jnp.zeros((ZR, D), jnp.float32)
    batch_r = batch.reshape(NB, 1, BM)

    params = [(W0, b0, eps0, g0, be0),
              (W1, b1, eps1, g1, be1),
              (W2, b2, eps2, g2, be2)]
    h = x
    for li, (W, b, eps, g, be) in enumerate(params):
        parts = _sc_agg(src, dst, h, zeros_rows).reshape(2, NPAD, D)
        args = (h, parts, W, b.reshape(1, D), eps.reshape(1, 1),
                g.reshape(1, D), be.reshape(1, D))
        if li < 2:
            h = _layer(*args)
        else:
            out = _layer_pool(*args, batch_r)
    return out


# speedup vs baseline: 1.2889x; 1.0118x over previous
import functools

import jax
import jax.numpy as jnp
from jax import lax
from jax.experimental import pallas as pl
from jax.experimental.pallas import tpu as pltpu
from jax.experimental.pallas import tpu_sc as plsc

N = 10000
E = 320000
D = 128
G = 64

NC = 2
NS = 16
NW = NC * NS
CH = 80
C = 125
SUP = 4
SCH = CH // SUP
NPAD = 10240
ZR = 80
RPT = NPAD // NS

BM = 1000
NB = N // BM


NZ = RPT // ZR


def _sc_agg_body(src_hbm, dst_hbm, h_hbm, z_hbm, out_hbm,
                 agg_sh, idx_s0, idx_d0, idx_s1, idx_d1,
                 rows0, rows1,
                 sem0, sem1, semI0, semI1, semR0, semR1, semW0, semW1):
    c = lax.axis_index("c")
    s = lax.axis_index("s")
    wid = c * NS + s
    zb0 = rows0.at[pl.ds(0, ZR)]
    zb1 = rows1.at[pl.ds(0, ZR)]

    pltpu.async_copy(z_hbm, zb0, semR0)
    pltpu.async_copy(src_hbm.at[wid, 0], idx_s0, semI0)
    pltpu.async_copy(dst_hbm.at[wid, 0], idx_d0, semI0)
    pltpu.make_async_copy(z_hbm, zb0, semR0).wait()
    for k in range(NZ):
        pltpu.async_copy(zb0, agg_sh.at[pl.ds((s * NZ + k) * ZR, ZR)], semW0)
    pltpu.make_async_copy(src_hbm.at[wid, 0], idx_s0, semI0).wait()
    pltpu.make_async_copy(dst_hbm.at[wid, 0], idx_d0, semI0).wait()
    for k in range(NZ):
        pltpu.make_async_copy(zb0, agg_sh.at[pl.ds(0, ZR)], semW0).wait()
    pltpu.async_copy(h_hbm.at[idx_s0.at[0]], rows0, sem0)
    plsc.subcore_barrier()

    idx_pairs = [(idx_s0, idx_d0, semI0), (idx_s1, idx_d1, semI1)]
    for t in range(SUP):
        cs, cd, csem = idx_pairs[t % 2]
        if t > 0:
            pltpu.make_async_copy(src_hbm.at[wid, t], cs, csem).wait()
            pltpu.make_async_copy(dst_hbm.at[wid, t], cd, csem).wait()
        if t + 1 < SUP:
            ns, nd, nsem = idx_pairs[(t + 1) % 2]
            pltpu.async_copy(src_hbm.at[wid, t + 1], ns, nsem)
            pltpu.async_copy(dst_hbm.at[wid, t + 1], nd, nsem)

        def pair(g, carry, cs=cs, cd=cd):
            j = 2 * g
            pltpu.async_copy(h_hbm.at[cs.at[j + 1]], rows1, sem1)
            pltpu.make_async_copy(h_hbm.at[cs.at[j]], rows0, sem0).wait()
            pltpu.sync_copy(rows0, agg_sh.at[cd.at[j]], add=True)
            pltpu.async_copy(h_hbm.at[cs.at[j + 2]], rows0, sem0)
            pltpu.make_async_copy(h_hbm.at[cs.at[j + 1]], rows1, sem1).wait()
            pltpu.sync_copy(rows1, agg_sh.at[cd.at[j + 1]], add=True)
            return carry

        if t > 0:
            pltpu.async_copy(h_hbm.at[cs.at[0]], rows0, sem0)
        lax.fori_loop(0, (SCH - 1) // 2, pair, 0)
        if SCH % 2 == 1:
            pltpu.make_async_copy(h_hbm.at[cs.at[SCH - 1]], rows0,
                                  sem0).wait()
            pltpu.sync_copy(rows0, agg_sh.at[cd.at[SCH - 1]], add=True)
        else:
            pltpu.async_copy(h_hbm.at[cs.at[SCH - 1]], rows1, sem1)
            pltpu.make_async_copy(h_hbm.at[cs.at[SCH - 2]], rows0,
                                  sem0).wait()
            pltpu.sync_copy(rows0, agg_sh.at[cd.at[SCH - 2]], add=True)
            pltpu.make_async_copy(h_hbm.at[cs.at[SCH - 1]], rows1,
                                  sem1).wait()
            pltpu.sync_copy(rows1, agg_sh.at[cd.at[SCH - 1]], add=True)
    plsc.subcore_barrier()

    bufs = (zb0, zb1)
    rsems = (semR0, semR1)
    wsems = (semW0, semW1)
    pltpu.async_copy(agg_sh.at[pl.ds(s * NZ * ZR, ZR)], zb0, semR0)
    for k in range(NZ):
        kb = k % 2
        cur = bufs[kb]
        r0 = (s * NZ + k) * ZR
        pltpu.make_async_copy(agg_sh.at[pl.ds(r0, ZR)], cur, rsems[kb]).wait()
        if k >= 1:
            pltpu.make_async_copy(bufs[1 - kb], out_hbm.at[pl.ds(0, ZR)],
                                  wsems[1 - kb]).wait()
        if k + 1 < NZ:
            r1 = (s * NZ + k + 1) * ZR
            pltpu.async_copy(agg_sh.at[pl.ds(r1, ZR)], bufs[1 - kb],
                             rsems[1 - kb])
        pltpu.async_copy(cur, out_hbm.at[pl.ds(c * NPAD + r0, ZR)], wsems[kb])
    pltpu.make_async_copy(bufs[(NZ - 1) % 2], out_hbm.at[pl.ds(0, ZR)],
                          wsems[(NZ - 1) % 2]).wait()


_sc_agg = functools.partial(
    pl.kernel,
    mesh=plsc.VectorSubcoreMesh(core_axis_name="c", subcore_axis_name="s"),
    out_type=jax.ShapeDtypeStruct((2 * NPAD, D), jnp.float32),
    scratch_types=[
        pltpu.VMEM_SHARED((NPAD, D), jnp.float32),
        pltpu.VMEM((SCH, C), jnp.int32),
        pltpu.VMEM((SCH, C), jnp.int32),
        pltpu.VMEM((SCH, C), jnp.int32),
        pltpu.VMEM((SCH, C), jnp.int32),
        pltpu.VMEM((C, D), jnp.float32),
        pltpu.VMEM((C, D), jnp.float32),
        pltpu.SemaphoreType.DMA,
        pltpu.SemaphoreType.DMA,
        pltpu.SemaphoreType.DMA,
        pltpu.SemaphoreType.DMA,
        pltpu.SemaphoreType.DMA,
        pltpu.SemaphoreType.DMA,
        pltpu.SemaphoreType.DMA,
        pltpu.SemaphoreType.DMA,
    ],
)(_sc_agg_body)


def _phase0(h_ref, p_ref, w_ref, b_ref, eps_ref, y_s, st_s, i):
    z = h_ref[...] * (1.0 + eps_ref[0, 0]) + p_ref[0] + p_ref[1]
    y = jnp.dot(z, w_ref[...], preferred_element_type=jnp.float32) + b_ref[...]
    y_s[pl.ds(i * BM, BM), :] = y
    s1 = jnp.sum(y, axis=0, keepdims=True)
    s2 = jnp.sum(y * y, axis=0, keepdims=True)
    upd = jnp.concatenate([s1, s2, jnp.zeros((6, D), jnp.float32)], axis=0)

    @pl.when(i == 0)
    def _():
        st_s[...] = upd

    @pl.when(i > 0)
    def _():
        st_s[...] = st_s[...] + upd


def _bn_relu_from_scratch(y_s, st_s, g_ref, be_ref, j):
    y = y_s[pl.ds(j * BM, BM), :]
    mean = st_s[0:1, :] * (1.0 / N)
    ex2 = st_s[1:2, :] * (1.0 / N)
    var = ex2 - mean * mean
    inv = lax.rsqrt(var + 1e-5) * g_ref[...]
    return jnp.maximum((y - mean) * inv + be_ref[...], 0.0)


def _layer_body(h_ref, p_ref, w_ref, b_ref, eps_ref, g_ref, be_ref,
                o_ref, y_s, st_s):
    i = pl.program_id(0)

    @pl.when(i < NB)
    def _():
        _phase0(h_ref, p_ref, w_ref, b_ref, eps_ref, y_s, st_s, i)

    @pl.when(i >= NB)
    def _():
        o_ref[...] = _bn_relu_from_scratch(y_s, st_s, g_ref, be_ref, i - NB)


_COMMON_SPECS = [
    pl.BlockSpec((BM, D), lambda i: (jnp.minimum(i, NB - 1), 0)),
    pl.BlockSpec((2, BM, D), lambda i: (0, jnp.minimum(i, NB - 1), 0)),
    pl.BlockSpec((D, D), lambda i: (0, 0)),
    pl.BlockSpec((1, D), lambda i: (0, 0)),
    pl.BlockSpec(memory_space=pltpu.SMEM),
    pl.BlockSpec((1, D), lambda i: (0, 0)),
    pl.BlockSpec((1, D), lambda i: (0, 0)),
]


def _layer(h, parts, w, b2, eps2, g2, be2):
    return pl.pallas_call(
        _layer_body,
        grid=(2 * NB,),
        in_specs=_COMMON_SPECS,
        out_specs=pl.BlockSpec((BM, D), lambda i: (jnp.maximum(i - NB, 0), 0)),
        out_shape=jax.ShapeDtypeStruct((N, D), jnp.float32),
        scratch_shapes=[
            pltpu.VMEM((N, D), jnp.float32),
            pltpu.VMEM((8, D), jnp.float32),
        ],
    )(h, parts, w, b2, eps2, g2, be2)


def _layer_pool_body(h_ref, p_ref, w_ref, b_ref, eps_ref, g_ref, be_ref,
                     bt_ref, o_ref, y_s, st_s, sums, cnts):
    i = pl.program_id(0)

    @pl.when(i < NB)
    def _():
        _phase0(h_ref, p_ref, w_ref, b_ref, eps_ref, y_s, st_s, i)

    @pl.when(i >= NB)
    def _():
        h = _bn_relu_from_scratch(y_s, st_s, g_ref, be_ref, i - NB)
        bvec = bt_ref[0, 0, :]
        oh = (bvec[:, None] == lax.broadcasted_iota(jnp.int32, (BM, G), 1))
        oh = oh.astype(jnp.float32)
        psum = lax.dot_general(oh, h, (((0,), (0,)), ((), ())),
                               preferred_element_type=jnp.float32)
        pcnt = jnp.broadcast_to(jnp.sum(oh, axis=0)[:, None], (G, D))

        @pl.when(i == NB)
        def _():
            sums[...] = psum
            cnts[...] = pcnt

        @pl.when(i > NB)
        def _():
            sums[...] = sums[...] + psum
            cnts[...] = cnts[...] + pcnt

        @pl.when(i == 2 * NB - 1)
        def _():
            o_ref[...] = sums[...] / jnp.maximum(cnts[...], 1.0)


def _layer_pool(h, parts, w, b2, eps2, g2, be2, batch_r):
    return pl.pallas_call(
        _layer_pool_body,
        grid=(2 * NB,),
        in_specs=_COMMON_SPECS + [
            pl.BlockSpec((1, 1, BM), lambda i: (jnp.maximum(i - NB, 0), 0, 0)),
        ],
        out_specs=pl.BlockSpec((G, D), lambda i: (0, 0)),
        out_shape=jax.ShapeDtypeStruct((G, D), jnp.float32),
        scratch_shapes=[
            pltpu.VMEM((N, D), jnp.float32),
            pltpu.VMEM((8, D), jnp.float32),
            pltpu.VMEM((G, D), jnp.float32),
            pltpu.VMEM((G, D), jnp.float32),
        ],
    )(h, parts, w, b2, eps2, g2, be2, batch_r)


def kernel(edge_index, x, batch, W0, b0, eps0, g0, be0,
           W1, b1, eps1, g1, be1, W2, b2, eps2, g2, be2):
    src = edge_index[0].reshape(NW, SUP, SCH, C)
    dst = edge_index[1].reshape(NW, SUP, SCH, C)
    zeros_rows = jnp.zeros((ZR, D), jnp.float32)
    batch_r = batch.reshape(NB, 1, BM)

    params = [(W0, b0, eps0, g0, be0),
              (W1, b1, eps1, g1, be1),
              (W2, b2, eps2, g2, be2)]
    h = x
    for li, (W, b, eps, g, be) in enumerate(params):
        parts = _sc_agg(src, dst, h, zeros_rows).reshape(2, NPAD, D)
        args = (h, parts, W, b.reshape(1, D), eps.reshape(1, 1),
                g.reshape(1, D), be.reshape(1, D))
        if li < 2:
            h = _layer(*args)
        else:
            out = _layer_pool(*args, batch_r)
    return out
